# Initial kernel scaffold; baseline (speedup 1.0000x reference)
import jax
import jax.numpy as jnp
from jax.experimental import pallas as pl


def kernel(idx, layer_matrix):
    raise NotImplementedError("write your pallas kernel here")


# speedup vs baseline: 5.8042x; 5.8042x over previous
import functools

import jax
import jax.numpy as jnp
from jax import lax
from jax.experimental import pallas as pl
from jax.experimental.pallas import tpu as pltpu
from jax.experimental.pallas import tpu_sc as plsc

N_TAGS = 64
BATCH = 16384
HIST = 200
N = BATCH * HIST
D = N_TAGS

NW = 32
SUB = 128
CHUNK = 512
K = CHUNK // SUB
SUPER = 4
IDXSUP = SUPER * CHUNK
N_W = N // NW
G = N_W // CHUNK
NSUP = N_W // IDXSUP
NPAIR = NSUP // 2

STORE_BYTES = CHUNK * D * 4


@functools.lru_cache(maxsize=None)
def _make_kernel():
    mesh = plsc.VectorSubcoreMesh(core_axis_name="c", subcore_axis_name="s")

    @functools.partial(
        pl.kernel,
        mesh=mesh,
        out_type=jax.ShapeDtypeStruct((N, D), jnp.float32),
        scratch_types=[
            pltpu.VMEM_SHARED((N_TAGS, D), jnp.float32),
            pltpu.VMEM((2, IDXSUP), jnp.int32),
            pltpu.VMEM((2, CHUNK, D), jnp.float32),
            pltpu.SemaphoreType.DMA,
            pltpu.SemaphoreType.DMA,
            pltpu.SemaphoreType.DMA,
            pltpu.SemaphoreType.DMA,
            pltpu.SemaphoreType.DMA,
        ],
        compiler_params=pltpu.CompilerParams(use_tc_tiling_on_sc=False),
    )
    def gather_kernel(idx_hbm, table_hbm, out_hbm, table_sp, idx_v, rows_v,
                      sem_i0, sem_i1, sem_g, sem_o0, sem_o1):
        cid = lax.axis_index("c")
        sid = lax.axis_index("s")
        wid = sid * 2 + cid
        base = wid * N_W
        sem_i = (sem_i0, sem_i1)
        sem_o = (sem_o0, sem_o1)

        @pl.when(wid < 2)
        def _():
            pltpu.sync_copy(table_hbm, table_sp)
        plsc.subcore_barrier()

        def idx_copy(s, slot):
            return pltpu.make_async_copy(
                idx_hbm.at[pl.ds(base + s * IDXSUP, IDXSUP)],
                idx_v.at[slot], sem_i[slot])

        def store_copy(off, b):
            return pltpu.make_async_copy(
                rows_v.at[b], out_hbm.at[pl.ds(off, CHUNK)], sem_o[b])

        idx_copy(0, 0).start()
        idx_copy(1, 1).start()

        def pair_body(sp, carry):
            for slot in range(2):
                s = 2 * sp + slot
                idx_copy(s, slot).wait()
                for c in range(SUPER):
                    b = c % 2
                    g = s * SUPER + c

                    if c >= 2:
                        store_copy(base + (g - 2) * CHUNK, b).wait()
                    else:
                        @pl.when(s > 0)
                        def _():
                            store_copy(base + (g - 2) * CHUNK, b).wait()

                    gathers = [
                        pltpu.async_copy(
                            table_sp.at[idx_v.at[slot].at[pl.ds(c * CHUNK + j * SUB, SUB)]],
                            rows_v.at[b].at[pl.ds(j * SUB, SUB)],
                            sem_g)
                        for j in range(K)
                    ]
                    for cp in gathers:
                        cp.wait()
                    store_copy(base + g * CHUNK, b).start()
                @pl.when(s + 2 < NSUP)
                def _():
                    idx_copy(s + 2, slot).start()
            return carry

        lax.fori_loop(0, NPAIR, pair_body, 0)

        store_copy(base + (G - 2) * CHUNK, 0).wait()
        store_copy(base + (G - 1) * CHUNK, 1).wait()

    return gather_kernel


def kernel(idx, layer_matrix):
    idx_flat = idx.reshape(N).astype(jnp.int32)
    out = _make_kernel()(idx_flat, layer_matrix)
    return out.reshape(BATCH, HIST, D)
